# Initial kernel scaffold; baseline (speedup 1.0000x reference)
import jax
import jax.numpy as jnp
from jax.experimental import pallas as pl


def kernel(h_author, h_term, h_paper, edge_index_author, edge_index_term, W_author, b_author, W_term, b_term):
    raise NotImplementedError("write your pallas kernel here")


# speedup vs baseline: 2.0487x; 2.0487x over previous
import functools

import jax
import jax.numpy as jnp
from jax import lax
from jax.experimental import pallas as pl
from jax.experimental.pallas import tpu as pltpu
from jax.experimental.pallas import tpu_sc as plsc

N_TAIL = 10000
N_TAR = 10000
D = 256
E = 160000

NC = 2
NS = 16
K = 128
HALF = N_TAR // NC
ACC_ROWS = 5120
E_TOT = 2 * E
E_PAD = 323584
CHUNKS = E_PAD // (NS * K)
ROWS_PER_TILE = 312


def _proj_body(h_ref, w_ref, b_ref, o_ref):
    x = lax.dot_general(
        h_ref[0], w_ref[0], (((1,), (1,)), ((), ())),
        preferred_element_type=jnp.float32,
        precision=lax.Precision.HIGHEST)
    o_ref[0] = x + b_ref[0]


def _project(hs, ws, bs):
    grid = (2, 10)
    blk = N_TAIL // 10
    return pl.pallas_call(
        _proj_body,
        grid=grid,
        in_specs=[
            pl.BlockSpec((1, blk, D), lambda m, i: (m, i, 0)),
            pl.BlockSpec((1, D, D), lambda m, i: (m, 0, 0)),
            pl.BlockSpec((1, 1, D), lambda m, i: (m, 0, 0)),
        ],
        out_specs=pl.BlockSpec((1, blk, D), lambda m, i: (m, i, 0)),
        out_shape=jax.ShapeDtypeStruct((2, N_TAIL, D), jnp.float32),
    )(hs, ws, bs)


def _sc_body(table, src, dst, out, srcbuf, dstbuf, locbuf, rows, zbuf, acc, sem):
    c = lax.axis_index("c")
    s = lax.axis_index("s")
    lo = c * HALF

    def zrow(i, _):
        r = i // 16
        j = i % 16
        zbuf[r, pl.ds(j * 16, 16)] = jnp.zeros((16,), jnp.float32)
        return 0
    lax.fori_loop(0, 16 * 16, zrow, 0)

    zbase = s * (ACC_ROWS // NS)
    def zacc(i, _):
        pltpu.sync_copy(zbuf, acc.at[pl.ds(zbase + i * 16, 16)])
        return 0
    lax.fori_loop(0, ACC_ROWS // NS // 16, zacc, 0)

    plsc.subcore_barrier()

    tile_base = s * (CHUNKS * K)

    def chunk(g, _):
        base = tile_base + g * K
        pltpu.sync_copy(src.at[pl.ds(base, K)], srcbuf)
        pltpu.sync_copy(dst.at[pl.ds(base, K)], dstbuf)
        for j in range(K // 16):
            d = dstbuf[pl.ds(j * 16, 16)]
            ok = (d >= lo) & (d < lo + HALF)
            locbuf[pl.ds(j * 16, 16)] = jnp.where(ok, d - lo, HALF)
        pltpu.async_copy(table.at[srcbuf], rows, sem).wait()
        pltpu.async_copy(rows, acc.at[locbuf], sem, add=True).wait()
        return 0

    lax.fori_loop(0, CHUNKS, chunk, 0)

    plsc.subcore_barrier()

    wbase = s * ROWS_PER_TILE
    for t in range(3):
        n = 104
        r0 = wbase + t * n
        pltpu.sync_copy(acc.at[pl.ds(r0, n)], rows.at[pl.ds(0, n)])
        pltpu.sync_copy(rows.at[pl.ds(0, n)], out.at[pl.ds(lo + r0, n)])

    @pl.when(s == 0)
    def _():
        r0 = NS * ROWS_PER_TILE
        n = HALF - r0
        pltpu.sync_copy(acc.at[pl.ds(r0, n)], rows.at[pl.ds(0, n)])
        pltpu.sync_copy(rows.at[pl.ds(0, n)], out.at[pl.ds(lo + r0, n)])


_sc_agg = functools.partial(
    pl.kernel,
    out_type=jax.ShapeDtypeStruct((N_TAR, D), jnp.float32),
    mesh=plsc.VectorSubcoreMesh(core_axis_name="c", subcore_axis_name="s"),
    scratch_types=[
        pltpu.VMEM((K,), jnp.int32),
        pltpu.VMEM((K,), jnp.int32),
        pltpu.VMEM((K,), jnp.int32),
        pltpu.VMEM((K, D), jnp.float32),
        pltpu.VMEM((16, D), jnp.float32),
        pltpu.VMEM_SHARED((ACC_ROWS, D), jnp.float32),
        pltpu.SemaphoreType.DMA,
    ],
    compiler_params=pltpu.CompilerParams(use_tc_tiling_on_sc=False),
)(_sc_body)


@jax.jit
def kernel(h_author, h_term, h_paper, edge_index_author, edge_index_term,
           W_author, b_author, W_term, b_term):
    hs = jnp.stack([h_author, h_term])
    ws = jnp.stack([W_author, W_term])
    bs = jnp.stack([b_author, b_term])[:, None, :]
    table = _project(hs, ws, bs).reshape(2 * N_TAIL, D)

    npad = E_PAD - E_TOT
    src = jnp.concatenate([
        edge_index_author[0], edge_index_term[0] + N_TAIL,
        jnp.zeros((npad,), jnp.int32)])
    dst = jnp.concatenate([
        edge_index_author[1], edge_index_term[1],
        jnp.full((npad,), N_TAR, jnp.int32)])

    return _sc_agg(table, src, dst)


# speedup vs baseline: 2.9943x; 1.4616x over previous
import functools

import jax
import jax.numpy as jnp
from jax import lax
from jax.experimental import pallas as pl
from jax.experimental.pallas import tpu as pltpu
from jax.experimental.pallas import tpu_sc as plsc

N_TAIL = 10000
N_TAR = 10000
D = 256
E = 160000

NC = 2
NS = 16
K = 128
HALF = N_TAR // NC
ACC_ROWS = 5120
E_TOT = 2 * E
CHUNKS = 160
E_PAD = CHUNKS * NS * K
GSENT = -1
ROWS_PER_TILE = 312


def _proj_body(h_ref, w_ref, b_ref, o_ref):
    x = lax.dot_general(
        h_ref[0], w_ref[0], (((1,), (1,)), ((), ())),
        preferred_element_type=jnp.float32,
        precision=lax.Precision.HIGHEST)
    o_ref[0] = x + b_ref[0]


def _project(hs, ws, bs):
    grid = (2, 10)
    blk = N_TAIL // 10
    return pl.pallas_call(
        _proj_body,
        grid=grid,
        in_specs=[
            pl.BlockSpec((1, blk, D), lambda m, i: (m, i, 0)),
            pl.BlockSpec((1, D, D), lambda m, i: (m, 0, 0)),
            pl.BlockSpec((1, 1, D), lambda m, i: (m, 0, 0)),
        ],
        out_specs=pl.BlockSpec((1, blk, D), lambda m, i: (m, i, 0)),
        out_shape=jax.ShapeDtypeStruct((2, N_TAIL, D), jnp.float32),
    )(hs, ws, bs)


def _sc_body(src2d, dst2d, table, out,
             rs0, rd0, rs1, rd1,
             fs0, fd0, fs1, fd1,
             rows0, rows1, zbuf, acc,
             si0, si1, sg0, sg1, ss):
    c = lax.axis_index("c")
    s = lax.axis_index("s")
    lo = c * HALF

    def zrow(i, _):
        r = i // 16
        j = i % 16
        zbuf[r, pl.ds(j * 16, 16)] = jnp.zeros((16,), jnp.float32)
        return 0
    lax.fori_loop(0, 16 * 16, zrow, 0)

    zbase = s * (ACC_ROWS // NS)
    def zacc(i, _):
        pltpu.sync_copy(zbuf, acc.at[pl.ds(zbase + i * 16, 16)])
        return 0
    lax.fori_loop(0, ACC_ROWS // NS // 16, zacc, 0)

    plsc.subcore_barrier()

    row_base = s * CHUNKS

    def remap(rs, rd, fs, fd):
        for j in range(K // 16):
            sl = pl.ds(j * 16, 16)
            sv = rs[sl]
            dv = rd[sl]
            ok = (dv >= lo) & (dv < lo + HALF)
            fs[sl] = jnp.where(ok, sv, GSENT)
            fd[sl] = jnp.where(ok, dv - lo, HALF)

    def fire_idx(g, rs, rd, si):
        base = (row_base + g) * K
        pltpu.async_copy(src2d.at[pl.ds(base, K)], rs, si)
        pltpu.async_copy(dst2d.at[pl.ds(base, K)], rd, si)

    def wait_idx(g, rs, rd, si):
        base = (row_base + g) * K
        pltpu.make_async_copy(src2d.at[pl.ds(base, K)], rs, si).wait()
        pltpu.make_async_copy(dst2d.at[pl.ds(base, K)], rd, si).wait()

    def fire_gather(fs, rows, sg):
        pltpu.async_copy(
            table.at[plsc.Indices(fs, ignored_value=GSENT)], rows, sg)

    def wait_gather(fs, rows, sg):
        pltpu.make_async_copy(
            table.at[plsc.Indices(fs, ignored_value=GSENT)], rows, sg).wait()

    def scatter(fd, rows):
        pltpu.async_copy(
            rows, acc.at[plsc.Indices(fd, ignored_value=HALF)], ss,
            add=True).wait()

    def chunk(g, _):
        fire_idx(g, rs0, rd0, si0)
        wait_idx(g, rs0, rd0, si0)
        remap(rs0, rd0, fs0, fd0)
        fire_gather(fs0, rows0, sg0)
        wait_gather(fs0, rows0, sg0)
        scatter(fd0, rows0)
        return 0

    lax.fori_loop(0, CHUNKS, chunk, 0)

    plsc.subcore_barrier()

    wbase = s * ROWS_PER_TILE
    for t in range(3):
        n = 104
        r0 = wbase + t * n
        pltpu.sync_copy(acc.at[pl.ds(r0, n)], rows0.at[pl.ds(0, n)])
        pltpu.sync_copy(rows0.at[pl.ds(0, n)], out.at[pl.ds(lo + r0, n)])

    @pl.when(s == 0)
    def _():
        r0 = NS * ROWS_PER_TILE
        n = HALF - r0
        pltpu.sync_copy(acc.at[pl.ds(r0, n)], rows0.at[pl.ds(0, n)])
        pltpu.sync_copy(rows0.at[pl.ds(0, n)], out.at[pl.ds(lo + r0, n)])


_sc_agg = functools.partial(
    pl.kernel,
    out_type=jax.ShapeDtypeStruct((N_TAR, D), jnp.float32),
    mesh=plsc.VectorSubcoreMesh(core_axis_name="c", subcore_axis_name="s"),
    scratch_types=[
        pltpu.VMEM((K,), jnp.int32),
        pltpu.VMEM((K,), jnp.int32),
        pltpu.VMEM((K,), jnp.int32),
        pltpu.VMEM((K,), jnp.int32),
        pltpu.VMEM((K,), jnp.int32),
        pltpu.VMEM((K,), jnp.int32),
        pltpu.VMEM((K,), jnp.int32),
        pltpu.VMEM((K,), jnp.int32),
        pltpu.VMEM((K, D), jnp.float32),
        pltpu.VMEM((K, D), jnp.float32),
        pltpu.VMEM((16, D), jnp.float32),
        pltpu.VMEM_SHARED((ACC_ROWS, D), jnp.float32),
        pltpu.SemaphoreType.DMA,
        pltpu.SemaphoreType.DMA,
        pltpu.SemaphoreType.DMA,
        pltpu.SemaphoreType.DMA,
        pltpu.SemaphoreType.DMA,
    ],
    compiler_params=pltpu.CompilerParams(use_tc_tiling_on_sc=False),
)(_sc_body)


@jax.jit
def kernel(h_author, h_term, h_paper, edge_index_author, edge_index_term,
           W_author, b_author, W_term, b_term):
    hs = jnp.stack([h_author, h_term])
    ws = jnp.stack([W_author, W_term])
    bs = jnp.stack([b_author, b_term])[:, None, :]
    table = _project(hs, ws, bs).reshape(2 * N_TAIL, D)

    npad = E_PAD - E_TOT
    src = jnp.concatenate([
        edge_index_author[0], edge_index_term[0] + N_TAIL,
        jnp.zeros((npad,), jnp.int32)])
    dst = jnp.concatenate([
        edge_index_author[1], edge_index_term[1],
        jnp.full((npad,), N_TAR, jnp.int32)])

    return _sc_agg(src, dst, table)


# speedup vs baseline: 3.8438x; 1.2837x over previous
import functools

import jax
import jax.numpy as jnp
from jax import lax
from jax.experimental import pallas as pl
from jax.experimental.pallas import tpu as pltpu
from jax.experimental.pallas import tpu_sc as plsc

N_TAIL = 10000
N_TAR = 10000
D = 256
E = 160000

NC = 2
NS = 16
K = 96
HALF = N_TAR // NC
ACC_ROWS = 5008
E_TOT = 2 * E
CHUNKS = 210
E_PAD = CHUNKS * NS * K
GSENT = -1
ROWS_PER_TILE = 312


def _proj_body(h_ref, w_ref, b_ref, o_ref):
    x = lax.dot_general(
        h_ref[0], w_ref[0], (((1,), (1,)), ((), ())),
        preferred_element_type=jnp.float32,
        precision=lax.Precision.HIGHEST)
    o_ref[0] = x + b_ref[0]


def _project(hs, ws, bs):
    grid = (2, 10)
    blk = N_TAIL // 10
    return pl.pallas_call(
        _proj_body,
        grid=grid,
        in_specs=[
            pl.BlockSpec((1, blk, D), lambda m, i: (m, i, 0)),
            pl.BlockSpec((1, D, D), lambda m, i: (m, 0, 0)),
            pl.BlockSpec((1, 1, D), lambda m, i: (m, 0, 0)),
        ],
        out_specs=pl.BlockSpec((1, blk, D), lambda m, i: (m, i, 0)),
        out_shape=jax.ShapeDtypeStruct((2, N_TAIL, D), jnp.float32),
    )(hs, ws, bs)


def _sc_body(src2d, dst2d, table, out,
             rs0, rd0, rs1, rd1,
             fs0, fd0, fs1, fd1,
             rows0, rows1, acc,
             si0, si1, sg0, sg1, ss):
    c = lax.axis_index("c")
    s = lax.axis_index("s")
    lo = c * HALF

    def zrow(i, _):
        r = i // 16
        j = i % 16
        rows0[r, pl.ds(j * 16, 16)] = jnp.zeros((16,), jnp.float32)
        return 0
    lax.fori_loop(0, 16 * 16, zrow, 0)

    zbase = s * (ACC_ROWS // NS)
    def zacc(i, _):
        pltpu.sync_copy(rows0.at[pl.ds(0, 16)],
                        acc.at[pl.ds(zbase + i * 16, 16)])
        return 0
    lax.fori_loop(0, 19, zacc, 0)
    pltpu.sync_copy(rows0.at[pl.ds(0, 9)],
                    acc.at[pl.ds(zbase + 304, 9)])

    plsc.subcore_barrier()

    row_base = s * CHUNKS

    def remap(rs, rd, fs, fd):
        for j in range(K // 16):
            sl = pl.ds(j * 16, 16)
            sv = rs[sl]
            dv = rd[sl]
            ok = (dv >= lo) & (dv < lo + HALF)
            fs[sl] = jnp.where(ok, sv, GSENT)
            fd[sl] = jnp.where(ok, dv - lo, HALF)

    def fire_idx(g, rs, rd, si):
        base = (row_base + g) * K
        pltpu.async_copy(src2d.at[pl.ds(base, K)], rs, si)
        pltpu.async_copy(dst2d.at[pl.ds(base, K)], rd, si)

    def wait_idx(g, rs, rd, si):
        base = (row_base + g) * K
        pltpu.make_async_copy(src2d.at[pl.ds(base, K)], rs, si).wait()
        pltpu.make_async_copy(dst2d.at[pl.ds(base, K)], rd, si).wait()

    def fire_gather(fs, rows, sg):
        pltpu.async_copy(
            table.at[plsc.Indices(fs, ignored_value=GSENT)], rows, sg)

    def wait_gather(fs, rows, sg):
        pltpu.make_async_copy(
            table.at[plsc.Indices(fs, ignored_value=GSENT)], rows, sg).wait()

    def scatter(fd, rows):
        pltpu.async_copy(
            rows, acc.at[plsc.Indices(fd, ignored_value=HALF)], ss,
            add=True).wait()

    def pair(t, _):
        gA = 2 * t
        gB = 2 * t + 1
        fire_idx(gA, rs0, rd0, si0)
        fire_idx(gB, rs1, rd1, si1)
        wait_idx(gA, rs0, rd0, si0)
        remap(rs0, rd0, fs0, fd0)
        fire_gather(fs0, rows0, sg0)
        wait_idx(gB, rs1, rd1, si1)
        remap(rs1, rd1, fs1, fd1)
        fire_gather(fs1, rows1, sg1)
        wait_gather(fs0, rows0, sg0)
        scatter(fd0, rows0)
        wait_gather(fs1, rows1, sg1)
        scatter(fd1, rows1)
        return 0

    lax.fori_loop(0, CHUNKS // 2, pair, 0)

    plsc.subcore_barrier()

    wbase = s * ROWS_PER_TILE
    for t in range(4):
        n = 78
        r0 = wbase + t * n
        pltpu.sync_copy(acc.at[pl.ds(r0, n)], rows0.at[pl.ds(0, n)])
        pltpu.sync_copy(rows0.at[pl.ds(0, n)], out.at[pl.ds(lo + r0, n)])

    @pl.when(s == 0)
    def _():
        r0 = NS * ROWS_PER_TILE
        n = HALF - r0
        pltpu.sync_copy(acc.at[pl.ds(r0, n)], rows0.at[pl.ds(0, n)])
        pltpu.sync_copy(rows0.at[pl.ds(0, n)], out.at[pl.ds(lo + r0, n)])


_sc_agg = functools.partial(
    pl.kernel,
    out_type=jax.ShapeDtypeStruct((N_TAR, D), jnp.float32),
    mesh=plsc.VectorSubcoreMesh(core_axis_name="c", subcore_axis_name="s"),
    scratch_types=[
        pltpu.VMEM((K,), jnp.int32),
        pltpu.VMEM((K,), jnp.int32),
        pltpu.VMEM((K,), jnp.int32),
        pltpu.VMEM((K,), jnp.int32),
        pltpu.VMEM((K,), jnp.int32),
        pltpu.VMEM((K,), jnp.int32),
        pltpu.VMEM((K,), jnp.int32),
        pltpu.VMEM((K,), jnp.int32),
        pltpu.VMEM((K, D), jnp.float32),
        pltpu.VMEM((K, D), jnp.float32),
        pltpu.VMEM_SHARED((ACC_ROWS, D), jnp.float32),
        pltpu.SemaphoreType.DMA,
        pltpu.SemaphoreType.DMA,
        pltpu.SemaphoreType.DMA,
        pltpu.SemaphoreType.DMA,
        pltpu.SemaphoreType.DMA,
    ],
    compiler_params=pltpu.CompilerParams(use_tc_tiling_on_sc=False),
)(_sc_body)


@jax.jit
def kernel(h_author, h_term, h_paper, edge_index_author, edge_index_term,
           W_author, b_author, W_term, b_term):
    hs = jnp.stack([h_author, h_term])
    ws = jnp.stack([W_author, W_term])
    bs = jnp.stack([b_author, b_term])[:, None, :]
    table = _project(hs, ws, bs).reshape(2 * N_TAIL, D)

    npad = E_PAD - E_TOT
    src = jnp.concatenate([
        edge_index_author[0], edge_index_term[0] + N_TAIL,
        jnp.zeros((npad,), jnp.int32)])
    dst = jnp.concatenate([
        edge_index_author[1], edge_index_term[1],
        jnp.full((npad,), N_TAR, jnp.int32)])

    return _sc_agg(src, dst, table)


# speedup vs baseline: 4.1833x; 1.0883x over previous
import functools

import jax
import jax.numpy as jnp
from jax import lax
from jax.experimental import pallas as pl
from jax.experimental.pallas import tpu as pltpu
from jax.experimental.pallas import tpu_sc as plsc

N_TAIL = 10000
N_TAR = 10000
D = 256
E = 160000

NC = 2
NS = 16
K = 96
HALF = N_TAR // NC
ACC_ROWS = 5008
E_TOT = 2 * E
CHUNKS = 212
E_PAD = CHUNKS * NS * K
GSENT = -1
ROWS_PER_TILE = 312


def _proj_body(h_ref, w_ref, b_ref, o_ref):
    x = lax.dot_general(
        h_ref[0], w_ref[0], (((1,), (1,)), ((), ())),
        preferred_element_type=jnp.float32,
        precision=lax.Precision.HIGHEST)
    o_ref[0] = x + b_ref[0]


def _project(hs, ws, bs):
    grid = (2, 10)
    blk = N_TAIL // 10
    return pl.pallas_call(
        _proj_body,
        grid=grid,
        in_specs=[
            pl.BlockSpec((1, blk, D), lambda m, i: (m, i, 0)),
            pl.BlockSpec((1, D, D), lambda m, i: (m, 0, 0)),
            pl.BlockSpec((1, 1, D), lambda m, i: (m, 0, 0)),
        ],
        out_specs=pl.BlockSpec((1, blk, D), lambda m, i: (m, i, 0)),
        out_shape=jax.ShapeDtypeStruct((2, N_TAIL, D), jnp.float32),
    )(hs, ws, bs)


def _sc_body(src2d, dst2d, table, out,
             rsp0, rdp0, rsp1, rdp1,
             fs0, fd0, fs1, fd1,
             rows0, rows1, acc,
             si0, si1, sg0, sg1, ss):
    c = lax.axis_index("c")
    s = lax.axis_index("s")
    lo = c * HALF

    def zrow(i, _):
        r = i // 16
        j = i % 16
        rows0[r, pl.ds(j * 16, 16)] = jnp.zeros((16,), jnp.float32)
        return 0
    lax.fori_loop(0, 16 * 16, zrow, 0)

    zbase = s * (ACC_ROWS // NS)
    def zacc(i, _):
        pltpu.sync_copy(rows0.at[pl.ds(0, 16)],
                        acc.at[pl.ds(zbase + i * 16, 16)])
        return 0
    lax.fori_loop(0, 19, zacc, 0)
    pltpu.sync_copy(rows0.at[pl.ds(0, 9)],
                    acc.at[pl.ds(zbase + 304, 9)])

    plsc.subcore_barrier()

    elem_base = s * CHUNKS * K
    P = CHUNKS // 2

    def remap(rs, rd, off, fs, fd):
        for j in range(K // 16):
            slr = pl.ds(off + j * 16, 16)
            slf = pl.ds(j * 16, 16)
            sv = rs[slr]
            dv = rd[slr]
            ok = (dv >= lo) & (dv < lo + HALF)
            fs[slf] = jnp.where(ok, sv, GSENT)
            fd[slf] = jnp.where(ok, dv - lo, HALF)

    def fire_idx(p, rsp, rdp, si):
        base = elem_base + p * 2 * K
        pltpu.async_copy(src2d.at[pl.ds(base, 2 * K)], rsp, si)
        pltpu.async_copy(dst2d.at[pl.ds(base, 2 * K)], rdp, si)

    def wait_idx(p, rsp, rdp, si):
        base = elem_base + p * 2 * K
        pltpu.make_async_copy(src2d.at[pl.ds(base, 2 * K)], rsp, si).wait()
        pltpu.make_async_copy(dst2d.at[pl.ds(base, 2 * K)], rdp, si).wait()

    def fire_gather(fs, rows, sg):
        pltpu.async_copy(
            table.at[plsc.Indices(fs, ignored_value=GSENT)], rows, sg)

    def wait_gather(fs, rows, sg):
        pltpu.make_async_copy(
            table.at[plsc.Indices(fs, ignored_value=GSENT)], rows, sg).wait()

    def scatter(fd, rows):
        pltpu.async_copy(
            rows, acc.at[plsc.Indices(fd, ignored_value=HALF)], ss,
            add=True).wait()

    fire_idx(0, rsp0, rdp0, si0)
    fire_idx(1, rsp1, rdp1, si1)

    sets = ((rsp0, rdp0, si0), (rsp1, rdp1, si1))

    def pairs2(q, _):
        for u in (0, 1):
            p = 2 * q + u
            rsp, rdp, si = sets[u]
            wait_idx(p, rsp, rdp, si)
            remap(rsp, rdp, 0, fs0, fd0)
            fire_gather(fs0, rows0, sg0)
            remap(rsp, rdp, K, fs1, fd1)
            fire_gather(fs1, rows1, sg1)
            pnext = jnp.minimum(p + 2, P - 1)
            fire_idx(pnext, rsp, rdp, si)
            wait_gather(fs0, rows0, sg0)
            scatter(fd0, rows0)
            wait_gather(fs1, rows1, sg1)
            scatter(fd1, rows1)
        return 0

    lax.fori_loop(0, P // 2, pairs2, 0)

    wait_idx(P - 1, rsp0, rdp0, si0)
    wait_idx(P - 1, rsp1, rdp1, si1)

    plsc.subcore_barrier()

    wbase = s * ROWS_PER_TILE
    for t in range(4):
        n = 78
        r0 = wbase + t * n
        pltpu.sync_copy(acc.at[pl.ds(r0, n)], rows0.at[pl.ds(0, n)])
        pltpu.sync_copy(rows0.at[pl.ds(0, n)], out.at[pl.ds(lo + r0, n)])

    @pl.when(s == 0)
    def _():
        r0 = NS * ROWS_PER_TILE
        n = HALF - r0
        pltpu.sync_copy(acc.at[pl.ds(r0, n)], rows0.at[pl.ds(0, n)])
        pltpu.sync_copy(rows0.at[pl.ds(0, n)], out.at[pl.ds(lo + r0, n)])


_sc_agg = functools.partial(
    pl.kernel,
    out_type=jax.ShapeDtypeStruct((N_TAR, D), jnp.float32),
    mesh=plsc.VectorSubcoreMesh(core_axis_name="c", subcore_axis_name="s"),
    scratch_types=[
        pltpu.VMEM((2 * K,), jnp.int32),
        pltpu.VMEM((2 * K,), jnp.int32),
        pltpu.VMEM((2 * K,), jnp.int32),
        pltpu.VMEM((2 * K,), jnp.int32),
        pltpu.VMEM((K,), jnp.int32),
        pltpu.VMEM((K,), jnp.int32),
        pltpu.VMEM((K,), jnp.int32),
        pltpu.VMEM((K,), jnp.int32),
        pltpu.VMEM((K, D), jnp.float32),
        pltpu.VMEM((K, D), jnp.float32),
        pltpu.VMEM_SHARED((ACC_ROWS, D), jnp.float32),
        pltpu.SemaphoreType.DMA,
        pltpu.SemaphoreType.DMA,
        pltpu.SemaphoreType.DMA,
        pltpu.SemaphoreType.DMA,
        pltpu.SemaphoreType.DMA,
    ],
    compiler_params=pltpu.CompilerParams(use_tc_tiling_on_sc=False),
)(_sc_body)


@jax.jit
def kernel(h_author, h_term, h_paper, edge_index_author, edge_index_term,
           W_author, b_author, W_term, b_term):
    hs = jnp.stack([h_author, h_term])
    ws = jnp.stack([W_author, W_term])
    bs = jnp.stack([b_author, b_term])[:, None, :]
    table = _project(hs, ws, bs).reshape(2 * N_TAIL, D)

    npad = E_PAD - E_TOT
    src = jnp.concatenate([
        edge_index_author[0], edge_index_term[0] + N_TAIL,
        jnp.zeros((npad,), jnp.int32)])
    dst = jnp.concatenate([
        edge_index_author[1], edge_index_term[1],
        jnp.full((npad,), N_TAR, jnp.int32)])

    return _sc_agg(src, dst, table)


# speedup vs baseline: 5.3292x; 1.2739x over previous
import functools

import jax
import jax.numpy as jnp
from jax import lax
from jax.experimental import pallas as pl
from jax.experimental.pallas import tpu as pltpu
from jax.experimental.pallas import tpu_sc as plsc

N_TAIL = 10000
N_TAR = 10000
D = 256
E = 160000

NC = 2
NS = 16
K = 64
HALF = N_TAR // NC
ACC_ROWS = 5008
E_TOT = 2 * E
T = 106
CHUNKS = 3 * T
E_PAD = CHUNKS * NS * K
GSENT = -1
ROWS_PER_TILE = 312


def _proj_body(h_ref, w_ref, b_ref, o_ref):
    x = lax.dot_general(
        h_ref[0], w_ref[0], (((1,), (1,)), ((), ())),
        preferred_element_type=jnp.float32,
        precision=lax.Precision.HIGHEST)
    o_ref[0] = x + b_ref[0]


def _project(hs, ws, bs):
    grid = (2, 10)
    blk = N_TAIL // 10
    return pl.pallas_call(
        _proj_body,
        grid=grid,
        in_specs=[
            pl.BlockSpec((1, blk, D), lambda m, i: (m, i, 0)),
            pl.BlockSpec((1, D, D), lambda m, i: (m, 0, 0)),
            pl.BlockSpec((1, 1, D), lambda m, i: (m, 0, 0)),
        ],
        out_specs=pl.BlockSpec((1, blk, D), lambda m, i: (m, i, 0)),
        out_shape=jax.ShapeDtypeStruct((2, N_TAIL, D), jnp.float32),
    )(hs, ws, bs)


def _sc_body(src2d, dst2d, table, out,
             rsp0, rdp0, rsp1, rdp1,
             fs0, fd0, fs1, fd1, fs2, fd2,
             rows0, rows1, rows2, acc,
             si0, si1, sg0, sg1, sg2, ss):
    c = lax.axis_index("c")
    s = lax.axis_index("s")
    lo = c * HALF

    def zrow(i, _):
        r = i // 16
        j = i % 16
        rows0[r, pl.ds(j * 16, 16)] = jnp.zeros((16,), jnp.float32)
        return 0
    lax.fori_loop(0, 16 * 16, zrow, 0)

    zbase = s * (ACC_ROWS // NS)
    def zacc(i, _):
        pltpu.sync_copy(rows0.at[pl.ds(0, 16)],
                        acc.at[pl.ds(zbase + i * 16, 16)])
        return 0
    lax.fori_loop(0, 19, zacc, 0)
    pltpu.sync_copy(rows0.at[pl.ds(0, 9)],
                    acc.at[pl.ds(zbase + 304, 9)])

    plsc.subcore_barrier()

    elem_base = s * CHUNKS * K
    KT = 3 * K

    F = ((fs0, fd0, rows0, sg0), (fs1, fd1, rows1, sg1),
         (fs2, fd2, rows2, sg2))
    SETS = ((rsp0, rdp0, si0), (rsp1, rdp1, si1))

    def remap(rsp, rdp, i, fs, fd):
        for j in range(K // 16):
            slr = pl.ds(i * K + j * 16, 16)
            slf = pl.ds(j * 16, 16)
            sv = rsp[slr]
            dv = rdp[slr]
            ok = (dv >= lo) & (dv < lo + HALF)
            fs[slf] = jnp.where(ok, sv, GSENT)
            fd[slf] = jnp.where(ok, dv - lo, HALF)

    def fire_idx(t, rsp, rdp, si):
        base = elem_base + t * KT
        pltpu.async_copy(src2d.at[pl.ds(base, KT)], rsp, si)
        pltpu.async_copy(dst2d.at[pl.ds(base, KT)], rdp, si)

    def wait_idx(t, rsp, rdp, si):
        base = elem_base + t * KT
        pltpu.make_async_copy(src2d.at[pl.ds(base, KT)], rsp, si).wait()
        pltpu.make_async_copy(dst2d.at[pl.ds(base, KT)], rdp, si).wait()

    def fire_gather(fs, rows, sg):
        pltpu.async_copy(
            table.at[plsc.Indices(fs, ignored_value=GSENT)], rows, sg)

    def wait_gather(fs, rows, sg):
        pltpu.make_async_copy(
            table.at[plsc.Indices(fs, ignored_value=GSENT)], rows, sg).wait()

    def scatter(fd, rows):
        pltpu.async_copy(
            rows, acc.at[plsc.Indices(fd, ignored_value=HALF)], ss,
            add=True).wait()

    fire_idx(0, rsp0, rdp0, si0)
    fire_idx(1, rsp1, rdp1, si1)
    wait_idx(0, rsp0, rdp0, si0)
    for i in range(3):
        fs, fd, rows, sg = F[i]
        remap(rsp0, rdp0, i, fs, fd)
        fire_gather(fs, rows, sg)

    def two_triples(w, _):
        for par in (0, 1):
            t = 2 * w + par
            rsp_c, rdp_c, si_c = SETS[par]
            rsp_n, rdp_n, si_n = SETS[1 - par]
            tn = jnp.minimum(t + 1, T - 1)
            tn2 = jnp.minimum(t + 2, T - 1)
            wait_idx(tn, rsp_n, rdp_n, si_n)
            for i in range(3):
                fs, fd, rows, sg = F[i]
                wait_gather(fs, rows, sg)
                scatter(fd, rows)
                remap(rsp_n, rdp_n, i, fs, fd)
                fire_gather(fs, rows, sg)
            fire_idx(tn2, rsp_c, rdp_c, si_c)
        return 0

    lax.fori_loop(0, T // 2, two_triples, 0)

    for i in range(3):
        fs, fd, rows, sg = F[i]
        wait_gather(fs, rows, sg)
    wait_idx(T - 1, rsp1, rdp1, si1)

    plsc.subcore_barrier()

    wbase = s * ROWS_PER_TILE
    for t in range(6):
        n = 52
        r0 = wbase + t * n
        pltpu.sync_copy(acc.at[pl.ds(r0, n)], rows0.at[pl.ds(0, n)])
        pltpu.sync_copy(rows0.at[pl.ds(0, n)], out.at[pl.ds(lo + r0, n)])

    @pl.when(s == 0)
    def _():
        r0 = NS * ROWS_PER_TILE
        n = HALF - r0
        pltpu.sync_copy(acc.at[pl.ds(r0, n)], rows0.at[pl.ds(0, n)])
        pltpu.sync_copy(rows0.at[pl.ds(0, n)], out.at[pl.ds(lo + r0, n)])


_sc_agg = functools.partial(
    pl.kernel,
    out_type=jax.ShapeDtypeStruct((N_TAR, D), jnp.float32),
    mesh=plsc.VectorSubcoreMesh(core_axis_name="c", subcore_axis_name="s"),
    scratch_types=[
        pltpu.VMEM((3 * K,), jnp.int32),
        pltpu.VMEM((3 * K,), jnp.int32),
        pltpu.VMEM((3 * K,), jnp.int32),
        pltpu.VMEM((3 * K,), jnp.int32),
        pltpu.VMEM((K,), jnp.int32),
        pltpu.VMEM((K,), jnp.int32),
        pltpu.VMEM((K,), jnp.int32),
        pltpu.VMEM((K,), jnp.int32),
        pltpu.VMEM((K,), jnp.int32),
        pltpu.VMEM((K,), jnp.int32),
        pltpu.VMEM((K, D), jnp.float32),
        pltpu.VMEM((K, D), jnp.float32),
        pltpu.VMEM((K, D), jnp.float32),
        pltpu.VMEM_SHARED((ACC_ROWS, D), jnp.float32),
        pltpu.SemaphoreType.DMA,
        pltpu.SemaphoreType.DMA,
        pltpu.SemaphoreType.DMA,
        pltpu.SemaphoreType.DMA,
        pltpu.SemaphoreType.DMA,
        pltpu.SemaphoreType.DMA,
    ],
    compiler_params=pltpu.CompilerParams(use_tc_tiling_on_sc=False),
)(_sc_body)


@jax.jit
def kernel(h_author, h_term, h_paper, edge_index_author, edge_index_term,
           W_author, b_author, W_term, b_term):
    hs = jnp.stack([h_author, h_term])
    ws = jnp.stack([W_author, W_term])
    bs = jnp.stack([b_author, b_term])[:, None, :]
    table = _project(hs, ws, bs).reshape(2 * N_TAIL, D)

    npad = E_PAD - E_TOT
    src = jnp.concatenate([
        edge_index_author[0], edge_index_term[0] + N_TAIL,
        jnp.zeros((npad,), jnp.int32)])
    dst = jnp.concatenate([
        edge_index_author[1], edge_index_term[1],
        jnp.full((npad,), N_TAR, jnp.int32)])

    return _sc_agg(src, dst, table)


# speedup vs baseline: 5.5091x; 1.0338x over previous
import functools

import jax
import jax.numpy as jnp
from jax import lax
from jax.experimental import pallas as pl
from jax.experimental.pallas import tpu as pltpu
from jax.experimental.pallas import tpu_sc as plsc

N_TAIL = 10000
N_TAR = 10000
D = 256
E = 160000

NC = 2
NS = 16
K = 64
KT = 3 * K
HALF = N_TAR // NC
ACC_ROWS = 5008
EPT = E // NS
T = 54
GSENT = -1
ROWS_PER_TILE = 312


def _proj_body(h_ref, w_ref, b_ref, o_ref):
    x = lax.dot_general(
        h_ref[...], w_ref[...], (((1,), (1,)), ((), ())),
        preferred_element_type=jnp.float32,
        precision=lax.Precision.HIGHEST)
    o_ref[...] = x + b_ref[...]


def _project(h, w, b):
    return pl.pallas_call(
        _proj_body,
        grid=(10,),
        in_specs=[
            pl.BlockSpec((N_TAIL // 10, D), lambda i: (i, 0)),
            pl.BlockSpec((D, D), lambda i: (0, 0)),
            pl.BlockSpec((1, D), lambda i: (0, 0)),
        ],
        out_specs=pl.BlockSpec((N_TAIL // 10, D), lambda i: (i, 0)),
        out_shape=jax.ShapeDtypeStruct((N_TAIL, D), jnp.float32),
    )(h, w, b)


def _sc_body(ea, et, xa, xt, out,
             rsp0, rdp0, rsp1, rdp1,
             fs0, fd0, fs1, fd1, fs2, fd2,
             rows0, rows1, rows2, acc,
             si0, si1, sg0, sg1, sg2, ss):
    c = lax.axis_index("c")
    s = lax.axis_index("s")
    lo = c * HALF

    def zrow(i, _):
        r = i // 16
        j = i % 16
        rows0[r, pl.ds(j * 16, 16)] = jnp.zeros((16,), jnp.float32)
        return 0
    lax.fori_loop(0, 16 * 16, zrow, 0)

    zbase = s * (ACC_ROWS // NS)
    def zacc(i, _):
        pltpu.sync_copy(rows0.at[pl.ds(0, 16)],
                        acc.at[pl.ds(zbase + i * 16, 16)])
        return 0
    lax.fori_loop(0, 19, zacc, 0)
    pltpu.sync_copy(rows0.at[pl.ds(0, 9)],
                    acc.at[pl.ds(zbase + 304, 9)])

    plsc.subcore_barrier()

    elem_base = s * EPT
    F = ((fs0, fd0, rows0, sg0), (fs1, fd1, rows1, sg1),
         (fs2, fd2, rows2, sg2))
    SETS = ((rsp0, rdp0, si0), (rsp1, rdp1, si1))

    def phase(e2d, table):
        def idx_base(t):
            return jnp.minimum(elem_base + t * KT, E - KT)

        def fire_idx(t, rsp, rdp, si):
            base = idx_base(t)
            pltpu.async_copy(e2d.at[0, pl.ds(base, KT)], rsp, si)
            pltpu.async_copy(e2d.at[1, pl.ds(base, KT)], rdp, si)

        def wait_idx(t, rsp, rdp, si):
            base = idx_base(t)
            pltpu.make_async_copy(e2d.at[0, pl.ds(base, KT)], rsp, si).wait()
            pltpu.make_async_copy(e2d.at[1, pl.ds(base, KT)], rdp, si).wait()

        def remap(t, rsp, rdp, i, fs, fd):
            pos0 = idx_base(t) + i * K
            want0 = elem_base + t * KT + i * K
            hi = elem_base + EPT
            for j in range(K // 16):
                slr = pl.ds(i * K + j * 16, 16)
                slf = pl.ds(j * 16, 16)
                pos = pos0 + j * 16 + lax.iota(jnp.int32, 16)
                sv = rsp[slr]
                dv = rdp[slr]
                ok = ((dv >= lo) & (dv < lo + HALF)
                      & (pos >= want0) & (pos < hi))
                fs[slf] = jnp.where(ok, sv, GSENT)
                fd[slf] = jnp.where(ok, dv - lo, HALF)

        def fire_gather(fs, rows, sg):
            pltpu.async_copy(
                table.at[plsc.Indices(fs, ignored_value=GSENT)], rows, sg)

        def wait_gather(fs, rows, sg):
            pltpu.make_async_copy(
                table.at[plsc.Indices(fs, ignored_value=GSENT)], rows,
                sg).wait()

        def scatter(fd, rows):
            pltpu.async_copy(
                rows, acc.at[plsc.Indices(fd, ignored_value=HALF)], ss,
                add=True).wait()

        fire_idx(0, rsp0, rdp0, si0)
        fire_idx(1, rsp1, rdp1, si1)
        wait_idx(0, rsp0, rdp0, si0)
        for i in range(3):
            fs, fd, rows, sg = F[i]
            remap(0, rsp0, rdp0, i, fs, fd)
            fire_gather(fs, rows, sg)

        def two_triples(w, _):
            for par in (0, 1):
                t = 2 * w + par
                rsp_c, rdp_c, si_c = SETS[par]
                rsp_n, rdp_n, si_n = SETS[1 - par]
                tn = jnp.minimum(t + 1, T - 1)
                tn2 = jnp.minimum(t + 2, T - 1)
                wait_idx(tn, rsp_n, rdp_n, si_n)
                for i in range(3):
                    fs, fd, rows, sg = F[i]
                    wait_gather(fs, rows, sg)
                    scatter(fd, rows)
                    remap(tn, rsp_n, rdp_n, i, fs, fd)
                    fire_gather(fs, rows, sg)
                fire_idx(tn2, rsp_c, rdp_c, si_c)
            return 0

        lax.fori_loop(0, T // 2, two_triples, 0)

        for i in range(3):
            fs, fd, rows, sg = F[i]
            wait_gather(fs, rows, sg)
        wait_idx(T - 1, rsp1, rdp1, si1)

    phase(ea, xa)
    phase(et, xt)

    plsc.subcore_barrier()

    wbase = s * ROWS_PER_TILE
    for t in range(6):
        n = 52
        r0 = wbase + t * n
        pltpu.sync_copy(acc.at[pl.ds(r0, n)], rows0.at[pl.ds(0, n)])
        pltpu.sync_copy(rows0.at[pl.ds(0, n)], out.at[pl.ds(lo + r0, n)])

    @pl.when(s == 0)
    def _():
        r0 = NS * ROWS_PER_TILE
        n = HALF - r0
        pltpu.sync_copy(acc.at[pl.ds(r0, n)], rows0.at[pl.ds(0, n)])
        pltpu.sync_copy(rows0.at[pl.ds(0, n)], out.at[pl.ds(lo + r0, n)])


_sc_agg = functools.partial(
    pl.kernel,
    out_type=jax.ShapeDtypeStruct((N_TAR, D), jnp.float32),
    mesh=plsc.VectorSubcoreMesh(core_axis_name="c", subcore_axis_name="s"),
    scratch_types=[
        pltpu.VMEM((KT,), jnp.int32),
        pltpu.VMEM((KT,), jnp.int32),
        pltpu.VMEM((KT,), jnp.int32),
        pltpu.VMEM((KT,), jnp.int32),
        pltpu.VMEM((K,), jnp.int32),
        pltpu.VMEM((K,), jnp.int32),
        pltpu.VMEM((K,), jnp.int32),
        pltpu.VMEM((K,), jnp.int32),
        pltpu.VMEM((K,), jnp.int32),
        pltpu.VMEM((K,), jnp.int32),
        pltpu.VMEM((K, D), jnp.float32),
        pltpu.VMEM((K, D), jnp.float32),
        pltpu.VMEM((K, D), jnp.float32),
        pltpu.VMEM_SHARED((ACC_ROWS, D), jnp.float32),
        pltpu.SemaphoreType.DMA,
        pltpu.SemaphoreType.DMA,
        pltpu.SemaphoreType.DMA,
        pltpu.SemaphoreType.DMA,
        pltpu.SemaphoreType.DMA,
        pltpu.SemaphoreType.DMA,
    ],
    compiler_params=pltpu.CompilerParams(use_tc_tiling_on_sc=False),
)(_sc_body)


@jax.jit
def kernel(h_author, h_term, h_paper, edge_index_author, edge_index_term,
           W_author, b_author, W_term, b_term):
    xa = _project(h_author, W_author, b_author[None, :])
    xt = _project(h_term, W_term, b_term[None, :])
    return _sc_agg(edge_index_author, edge_index_term, xa, xt)


# speedup vs baseline: 5.5184x; 1.0017x over previous
import functools

import jax
import jax.numpy as jnp
from jax import lax
from jax.experimental import pallas as pl
from jax.experimental.pallas import tpu as pltpu
from jax.experimental.pallas import tpu_sc as plsc

N_TAIL = 10000
N_TAR = 10000
D = 256
E = 160000

NC = 2
NS = 16
K = 64
KT = 3 * K
HALF = N_TAR // NC
ACC_ROWS = 5008
EPT = E // NS
T = 54
GSENT = -1
ROWS_PER_TILE = 312


def _proj_body(h_ref, w_ref, b_ref, o_ref):
    x = lax.dot_general(
        h_ref[...], w_ref[...], (((1,), (1,)), ((), ())),
        preferred_element_type=jnp.float32,
        precision=lax.Precision.HIGHEST)
    o_ref[...] = x + b_ref[...]


def _project(h, w, b):
    return pl.pallas_call(
        _proj_body,
        grid=(10,),
        in_specs=[
            pl.BlockSpec((N_TAIL // 10, D), lambda i: (i, 0)),
            pl.BlockSpec((D, D), lambda i: (0, 0)),
            pl.BlockSpec((1, D), lambda i: (0, 0)),
        ],
        out_specs=pl.BlockSpec((N_TAIL // 10, D), lambda i: (i, 0)),
        out_shape=jax.ShapeDtypeStruct((N_TAIL, D), jnp.float32),
    )(h, w, b)


def _sc_body(ea, et, xa, xt, out,
             rsp0, rdp0, rsp1, rdp1,
             fs0, fd0, fs1, fd1, fs2, fd2,
             rows0, rows1, rows2, acc,
             si0, si1, sg0, sg1, sg2, ss):
    c = lax.axis_index("c")
    s = lax.axis_index("s")
    lo = c * HALF

    def zrow(i, _):
        r = i // 16
        j = i % 16
        rows0[r, pl.ds(j * 16, 16)] = jnp.zeros((16,), jnp.float32)
        return 0
    lax.fori_loop(0, 16 * 16, zrow, 0)

    zbase = s * (ACC_ROWS // NS)
    def zacc(i, _):
        pltpu.sync_copy(rows0.at[pl.ds(0, 16)],
                        acc.at[pl.ds(zbase + i * 16, 16)])
        return 0
    lax.fori_loop(0, 19, zacc, 0)
    pltpu.sync_copy(rows0.at[pl.ds(0, 9)],
                    acc.at[pl.ds(zbase + 304, 9)])

    plsc.subcore_barrier()

    elem_base = s * EPT
    F = ((fs0, fd0, rows0, sg0), (fs1, fd1, rows1, sg1),
         (fs2, fd2, rows2, sg2))
    SETS = ((rsp0, rdp0, si0), (rsp1, rdp1, si1))

    def phase(e2d, table):
        def idx_base(t):
            return jnp.minimum(elem_base + t * KT, E - KT)

        def fire_idx(t, rsp, rdp, si):
            base = idx_base(t)
            pltpu.async_copy(e2d.at[0, pl.ds(base, KT)], rsp, si)
            pltpu.async_copy(e2d.at[1, pl.ds(base, KT)], rdp, si)

        def wait_idx(t, rsp, rdp, si):
            base = idx_base(t)
            pltpu.make_async_copy(e2d.at[0, pl.ds(base, KT)], rsp, si).wait()
            pltpu.make_async_copy(e2d.at[1, pl.ds(base, KT)], rdp, si).wait()

        def remap(t, rsp, rdp, i, fs, fd):
            pos0 = idx_base(t) + i * K
            lo_t = jnp.minimum(elem_base + t * KT, elem_base + EPT)
            hi_t = jnp.minimum(elem_base + (t + 1) * KT, elem_base + EPT)
            for j in range(K // 16):
                slr = pl.ds(i * K + j * 16, 16)
                slf = pl.ds(j * 16, 16)
                pos = pos0 + j * 16 + lax.iota(jnp.int32, 16)
                sv = rsp[slr]
                dv = rdp[slr]
                ok = ((dv >= lo) & (dv < lo + HALF)
                      & (pos >= lo_t) & (pos < hi_t))
                fs[slf] = jnp.where(ok, sv, GSENT)
                fd[slf] = jnp.where(ok, dv - lo, HALF)

        def fire_gather(fs, rows, sg):
            pltpu.async_copy(
                table.at[plsc.Indices(fs, ignored_value=GSENT)], rows, sg)

        def wait_gather(fs, rows, sg):
            pltpu.make_async_copy(
                table.at[plsc.Indices(fs, ignored_value=GSENT)], rows,
                sg).wait()

        def scatter(fd, rows):
            pltpu.async_copy(
                rows, acc.at[plsc.Indices(fd, ignored_value=HALF)], ss,
                add=True).wait()

        fire_idx(0, rsp0, rdp0, si0)
        fire_idx(1, rsp1, rdp1, si1)
        wait_idx(0, rsp0, rdp0, si0)
        for i in range(3):
            fs, fd, rows, sg = F[i]
            remap(0, rsp0, rdp0, i, fs, fd)
            fire_gather(fs, rows, sg)

        def two_triples(w, _):
            for par in (0, 1):
                t = 2 * w + par
                rsp_c, rdp_c, si_c = SETS[par]
                rsp_n, rdp_n, si_n = SETS[1 - par]
                tn = jnp.minimum(t + 1, T - 1)
                tn2 = jnp.minimum(t + 2, T - 1)
                wait_idx(tn, rsp_n, rdp_n, si_n)
                for i in range(3):
                    fs, fd, rows, sg = F[i]
                    wait_gather(fs, rows, sg)
                    scatter(fd, rows)
                    remap(tn, rsp_n, rdp_n, i, fs, fd)
                    fire_gather(fs, rows, sg)
                fire_idx(tn2, rsp_c, rdp_c, si_c)
            return 0

        lax.fori_loop(0, T // 2, two_triples, 0)

        for i in range(3):
            fs, fd, rows, sg = F[i]
            wait_gather(fs, rows, sg)
        wait_idx(T - 1, rsp1, rdp1, si1)

    phase(ea, xa)
    phase(et, xt)

    plsc.subcore_barrier()

    wbase = s * ROWS_PER_TILE
    for t in range(6):
        n = 52
        r0 = wbase + t * n
        pltpu.sync_copy(acc.at[pl.ds(r0, n)], rows0.at[pl.ds(0, n)])
        pltpu.sync_copy(rows0.at[pl.ds(0, n)], out.at[pl.ds(lo + r0, n)])

    @pl.when(s == 0)
    def _():
        r0 = NS * ROWS_PER_TILE
        n = HALF - r0
        pltpu.sync_copy(acc.at[pl.ds(r0, n)], rows0.at[pl.ds(0, n)])
        pltpu.sync_copy(rows0.at[pl.ds(0, n)], out.at[pl.ds(lo + r0, n)])


_sc_agg = functools.partial(
    pl.kernel,
    out_type=jax.ShapeDtypeStruct((N_TAR, D), jnp.float32),
    mesh=plsc.VectorSubcoreMesh(core_axis_name="c", subcore_axis_name="s"),
    scratch_types=[
        pltpu.VMEM((KT,), jnp.int32),
        pltpu.VMEM((KT,), jnp.int32),
        pltpu.VMEM((KT,), jnp.int32),
        pltpu.VMEM((KT,), jnp.int32),
        pltpu.VMEM((K,), jnp.int32),
        pltpu.VMEM((K,), jnp.int32),
        pltpu.VMEM((K,), jnp.int32),
        pltpu.VMEM((K,), jnp.int32),
        pltpu.VMEM((K,), jnp.int32),
        pltpu.VMEM((K,), jnp.int32),
        pltpu.VMEM((K, D), jnp.float32),
        pltpu.VMEM((K, D), jnp.float32),
        pltpu.VMEM((K, D), jnp.float32),
        pltpu.VMEM_SHARED((ACC_ROWS, D), jnp.float32),
        pltpu.SemaphoreType.DMA,
        pltpu.SemaphoreType.DMA,
        pltpu.SemaphoreType.DMA,
        pltpu.SemaphoreType.DMA,
        pltpu.SemaphoreType.DMA,
        pltpu.SemaphoreType.DMA,
    ],
    compiler_params=pltpu.CompilerParams(use_tc_tiling_on_sc=False),
)(_sc_body)


@jax.jit
def kernel(h_author, h_term, h_paper, edge_index_author, edge_index_term,
           W_author, b_author, W_term, b_term):
    xa = _project(h_author, W_author, b_author[None, :])
    xt = _project(h_term, W_term, b_term[None, :])
    return _sc_agg(edge_index_author, edge_index_term, xa, xt)
